# Initial kernel scaffold; baseline (speedup 1.0000x reference)
import jax
import jax.numpy as jnp
from jax.experimental import pallas as pl


def kernel(embs, ratings, node_num):
    raise NotImplementedError("write your pallas kernel here")


# speedup vs baseline: 6.6455x; 6.6455x over previous
import functools

import jax
import jax.numpy as jnp
from jax import lax
from jax.experimental import pallas as pl
from jax.experimental.pallas import tpu as pltpu
from jax.experimental.pallas import tpu_sc as plsc

NC = 2
NS = 16
NW = NC * NS
L = 16


def _pass_a(embs, r0, r1, n_pad, C):
    E = r0.shape[0]
    D = embs.shape[1]
    e_per_w = E // NW
    n_chunks = e_per_w // C
    mesh = plsc.VectorSubcoreMesh(core_axis_name="c", subcore_axis_name="s")

    @functools.partial(
        pl.kernel,
        out_type=[
            jax.ShapeDtypeStruct((E,), jnp.float32),
            jax.ShapeDtypeStruct((NC, n_pad), jnp.float32),
        ],
        mesh=mesh,
        compiler_params=pltpu.CompilerParams(needs_layout_passes=False),
        scratch_types=[
            pltpu.VMEM((C,), jnp.int32),
            pltpu.VMEM((C,), jnp.int32),
            pltpu.VMEM((C, D), jnp.float32),
            pltpu.VMEM((C, D), jnp.float32),
            pltpu.VMEM((C,), jnp.float32),
            pltpu.VMEM((1024,), jnp.float32),
            pltpu.VMEM((L,), jnp.float32),
            pltpu.VMEM_SHARED((n_pad,), jnp.float32),
            pltpu.SemaphoreType.DMA,
            pltpu.SemaphoreType.DMA,
        ],
    )
    def body(embs_h, r0_h, r1_h, exps_h, rowsum_h,
             idx0_v, idx1_v, a_v, b_v, e_v, z_v, tmp_v, rs_sh, sem0, sem1):
        cid = lax.axis_index("c")
        sid = lax.axis_index("s")
        wid = sid * NC + cid
        lane = lax.broadcasted_iota(jnp.int32, (L,), 0)
        zero16 = jnp.zeros((L,), jnp.float32)

        def zbuf(i, _):
            z_v[pl.ds(i * L, L)] = zero16
            return 0
        lax.fori_loop(0, 1024 // L, zbuf, 0)

        @pl.when(sid == 0)
        def _():
            def zsh(i, _):
                pltpu.sync_copy(z_v, rs_sh.at[pl.ds(i * 1024, 1024)])
                return 0
            lax.fori_loop(0, n_pad // 1024, zsh, 0)
        plsc.subcore_barrier()

        def chunk(ci, _):
            base = pl.multiple_of(wid * e_per_w + ci * C, 8)
            pltpu.sync_copy(r0_h.at[pl.ds(base, C)], idx0_v)
            pltpu.sync_copy(r1_h.at[pl.ds(base, C)], idx1_v)
            cp_a = pltpu.async_copy(embs_h.at[idx0_v], a_v, sem0)
            cp_b = pltpu.async_copy(embs_h.at[idx1_v], b_v, sem1)
            cp_a.wait()
            cp_b.wait()
            def dots_g(g, _):
                tmp_v[pl.ds(0, L)] = zero16
                for k in range(L):
                    e = g * L + k
                    acc = zero16
                    for j in range(D // L):
                        acc = acc + (a_v[e, pl.ds(j * L, L)]
                                     * b_v[e, pl.ds(j * L, L)])
                    plsc.addupdate_scatter(tmp_v, [lane * 0 + k], acc)
                e_v[pl.ds(g * L, L)] = jnp.exp(tmp_v[pl.ds(0, L)])
                return 0
            lax.fori_loop(0, C // L, dots_g, 0)
            pltpu.sync_copy(e_v, exps_h.at[pl.ds(base, C)])
            pltpu.sync_copy(e_v, rs_sh.at[idx0_v], add=True)
            return 0
        lax.fori_loop(0, n_chunks, chunk, 0)

        plsc.subcore_barrier()

        @pl.when(sid == 0)
        def _():
            pltpu.sync_copy(rs_sh, rowsum_h.at[cid])

    return body(embs, r0, r1)


def _pass_b(embs, r0, r1, exps, rowsum_p, n_pad, C):
    E = r0.shape[0]
    D = embs.shape[1]
    e_per_w = E // NW
    n_chunks = e_per_w // C
    rows_per_tile = n_pad // NS
    mesh = plsc.VectorSubcoreMesh(core_axis_name="c", subcore_axis_name="s")

    @functools.partial(
        pl.kernel,
        out_type=jax.ShapeDtypeStruct((NC, n_pad, D), jnp.float32),
        mesh=mesh,
        compiler_params=pltpu.CompilerParams(needs_layout_passes=False),
        scratch_types=[
            pltpu.VMEM((C,), jnp.int32),
            pltpu.VMEM((C,), jnp.int32),
            pltpu.VMEM((C,), jnp.float32),
            pltpu.VMEM((C, D), jnp.float32),
            pltpu.VMEM((n_pad,), jnp.float32),
            pltpu.VMEM((1024,), jnp.float32),
            pltpu.VMEM((1024,), jnp.float32),
            pltpu.VMEM((C, D), jnp.float32),
            pltpu.VMEM_SHARED((n_pad, D), jnp.float32),
            pltpu.SemaphoreType.DMA,
        ],
    )
    def body(embs_h, r0_h, r1_h, exps_h, rowsum_h, outp_h,
             idx0_v, idx1_v, w_v, b_v, rsum_v, s0_v, s1_v, z_v, out_sh, sem0):
        cid = lax.axis_index("c")
        sid = lax.axis_index("s")
        wid = sid * NC + cid
        lane = lax.broadcasted_iota(jnp.int32, (L,), 0)
        zero16 = jnp.zeros((L,), jnp.float32)

        def zrow(r, _):
            for j in range(D // L):
                z_v[r, pl.ds(j * L, L)] = zero16
            return 0
        lax.fori_loop(0, C, zrow, 0)
        for k in range(rows_per_tile // C):
            pltpu.sync_copy(z_v, out_sh.at[pl.ds(sid * rows_per_tile + k * C, C)])

        def rsblk(i, _):
            pltpu.sync_copy(rowsum_h.at[0, pl.ds(i * 1024, 1024)], s0_v)
            pltpu.sync_copy(rowsum_h.at[1, pl.ds(i * 1024, 1024)], s1_v)

            def add16(j, _):
                rsum_v[pl.ds(i * 1024 + j * L, L)] = (
                    s0_v[pl.ds(j * L, L)] + s1_v[pl.ds(j * L, L)])
                return 0
            lax.fori_loop(0, 1024 // L, add16, 0)
            return 0
        lax.fori_loop(0, n_pad // 1024, rsblk, 0)
        plsc.subcore_barrier()

        def chunk(ci, _):
            base = pl.multiple_of(wid * e_per_w + ci * C, 8)
            pltpu.sync_copy(r0_h.at[pl.ds(base, C)], idx0_v)
            pltpu.sync_copy(r1_h.at[pl.ds(base, C)], idx1_v)
            pltpu.sync_copy(exps_h.at[pl.ds(base, C)], w_v)
            pltpu.async_copy(embs_h.at[idx1_v], b_v, sem0).wait()
            def scale_g(g, _):
                i0 = idx0_v[pl.ds(g * L, L)]
                s16 = plsc.load_gather(rsum_v, [i0])
                w16 = w_v[pl.ds(g * L, L)] / s16
                for k in range(L):
                    e = g * L + k
                    w = w16[k]
                    for j in range(D // L):
                        b_v[e, pl.ds(j * L, L)] = b_v[e, pl.ds(j * L, L)] * w
                return 0
            lax.fori_loop(0, C // L, scale_g, 0)
            pltpu.sync_copy(b_v, out_sh.at[idx0_v], add=True)
            return 0
        lax.fori_loop(0, n_chunks, chunk, 0)

        plsc.subcore_barrier()
        pltpu.sync_copy(out_sh.at[pl.ds(sid * rows_per_tile, rows_per_tile)],
                        outp_h.at[cid, pl.ds(sid * rows_per_tile, rows_per_tile)])

    return body(embs, r0, r1, exps, rowsum_p)


def _combine(p0, p1):
    N, D = p0.shape
    BS = 400

    def body(a_ref, b_ref, o_ref):
        o_ref[...] = a_ref[...] + b_ref[...]

    return pl.pallas_call(
        body,
        grid=(N // BS,),
        in_specs=[pl.BlockSpec((BS, D), lambda i: (i, 0)),
                  pl.BlockSpec((BS, D), lambda i: (i, 0))],
        out_specs=pl.BlockSpec((BS, D), lambda i: (i, 0)),
        out_shape=jax.ShapeDtypeStruct((N, D), jnp.float32),
    )(p0, p1)


def kernel(embs, ratings, node_num):
    del node_num
    N, D = embs.shape
    E = ratings.shape[0]
    assert E % NW == 0 and D % L == 0
    C = 80
    n_pad = ((N + 1023) // 1024) * 1024
    r0 = ratings[:, 0].astype(jnp.int32)
    r1 = ratings[:, 1].astype(jnp.int32)
    exps, rowsum_p = _pass_a(embs, r0, r1, n_pad, C)
    outp = _pass_b(embs, r0, r1, exps, rowsum_p, n_pad, C)
    return _combine(outp[0, :N], outp[1, :N])


# speedup vs baseline: 11.9327x; 1.7956x over previous
import functools

import jax
import jax.numpy as jnp
from jax import lax
from jax.experimental import pallas as pl
from jax.experimental.pallas import tpu as pltpu
from jax.experimental.pallas import tpu_sc as plsc

NC = 2
NS = 16
NW = NC * NS
L = 16


def _pass_a(embs, r0, r1, n_pad, C):
    E = r0.shape[0]
    D = embs.shape[1]
    e_per_w = E // NW
    n_chunks = e_per_w // C
    assert n_chunks % 2 == 1 and n_chunks >= 3
    mesh = plsc.VectorSubcoreMesh(core_axis_name="c", subcore_axis_name="s")

    @functools.partial(
        pl.kernel,
        out_type=[
            jax.ShapeDtypeStruct((E,), jnp.float32),
            jax.ShapeDtypeStruct((NC, n_pad), jnp.float32),
        ],
        mesh=mesh,
        compiler_params=pltpu.CompilerParams(needs_layout_passes=False),
        scratch_types=[
            pltpu.VMEM((C,), jnp.int32), pltpu.VMEM((C,), jnp.int32),
            pltpu.VMEM((C,), jnp.int32), pltpu.VMEM((C,), jnp.int32),
            pltpu.VMEM((C, D), jnp.float32), pltpu.VMEM((C, D), jnp.float32),
            pltpu.VMEM((C, D), jnp.float32), pltpu.VMEM((C, D), jnp.float32),
            pltpu.VMEM((C,), jnp.float32), pltpu.VMEM((C,), jnp.float32),
            pltpu.VMEM((1024,), jnp.float32),
            pltpu.VMEM((L,), jnp.float32),
            pltpu.VMEM_SHARED((n_pad,), jnp.float32),
        ] + [pltpu.SemaphoreType.DMA] * 12,
    )
    def body(embs_h, r0_h, r1_h, exps_h, rowsum_h,
             idx0_0, idx0_1, idx1_0, idx1_1, a_0, a_1, b_0, b_1, e_0, e_1,
             z_v, tmp_v, rs_sh,
             si0_0, si0_1, si1_0, si1_1, sa_0, sa_1, sb_0, sb_1,
             se_0, se_1, sr_0, sr_1):
        idx0 = [idx0_0, idx0_1]
        idx1 = [idx1_0, idx1_1]
        a_v = [a_0, a_1]
        b_v = [b_0, b_1]
        e_v = [e_0, e_1]
        si0 = [si0_0, si0_1]
        si1 = [si1_0, si1_1]
        sa = [sa_0, sa_1]
        sb = [sb_0, sb_1]
        se = [se_0, se_1]
        sr = [sr_0, sr_1]

        cid = lax.axis_index("c")
        sid = lax.axis_index("s")
        wid = sid * NC + cid
        lane = lax.broadcasted_iota(jnp.int32, (L,), 0)
        zero16 = jnp.zeros((L,), jnp.float32)

        def cbase(ci):
            return pl.multiple_of(wid * e_per_w + ci * C, 8)

        def issue_idx(ci, p):
            base = cbase(ci)
            pltpu.async_copy(r0_h.at[pl.ds(base, C)], idx0[p], si0[p])
            pltpu.async_copy(r1_h.at[pl.ds(base, C)], idx1[p], si1[p])

        def wait_idx(p):
            pltpu.make_async_copy(r0_h.at[pl.ds(0, C)], idx0[p], si0[p]).wait()
            pltpu.make_async_copy(r1_h.at[pl.ds(0, C)], idx1[p], si1[p]).wait()

        def issue_gather(p):
            pltpu.async_copy(embs_h.at[idx0[p]], a_v[p], sa[p])
            pltpu.async_copy(embs_h.at[idx1[p]], b_v[p], sb[p])

        def wait_gather(p):
            pltpu.make_async_copy(embs_h.at[idx0[p]], a_v[p], sa[p]).wait()
            pltpu.make_async_copy(embs_h.at[idx1[p]], b_v[p], sb[p]).wait()

        def compute(ci, p):
            def dots_g(g, _):
                tmp_v[pl.ds(0, L)] = zero16
                for k in range(L):
                    e = g * L + k
                    acc = zero16
                    for j in range(D // L):
                        acc = acc + (a_v[p][e, pl.ds(j * L, L)]
                                     * b_v[p][e, pl.ds(j * L, L)])
                    plsc.addupdate_scatter(tmp_v, [lane * 0 + k], acc)
                e_v[p][pl.ds(g * L, L)] = jnp.exp(tmp_v[pl.ds(0, L)])
                return 0
            lax.fori_loop(0, C // L, dots_g, 0)
            pltpu.async_copy(e_v[p], exps_h.at[pl.ds(cbase(ci), C)], se[p])
            pltpu.async_copy(e_v[p], rs_sh.at[idx0[p]], sr[p], add=True)

        def wait_exps(p):
            pltpu.make_async_copy(e_v[p], exps_h.at[pl.ds(0, C)], se[p]).wait()

        def wait_rsum(p):
            pltpu.make_async_copy(e_v[p], rs_sh.at[idx0[p]], sr[p]).wait()

        issue_idx(0, 0)
        issue_idx(1, 1)

        def zbuf(i, _):
            z_v[pl.ds(i * L, L)] = zero16
            return 0
        lax.fori_loop(0, 1024 // L, zbuf, 0)

        @pl.when(sid == 0)
        def _():
            def zsh(i, _):
                pltpu.sync_copy(z_v, rs_sh.at[pl.ds(i * 1024, 1024)])
                return 0
            lax.fori_loop(0, n_pad // 1024, zsh, 0)
        plsc.subcore_barrier()

        wait_idx(0)
        issue_gather(0)

        def stage(ci, p):
            q = 1 - p

            @pl.when(ci + 1 < n_chunks)
            def _():
                wait_idx(q)
                issue_gather(q)
            wait_gather(p)

            @pl.when(ci >= 2)
            def _():
                wait_exps(p)
            compute(ci, p)

            @pl.when(ci + 2 < n_chunks)
            def _():
                wait_rsum(p)
                issue_idx(ci + 2, p)

        def pair(i, _):
            stage(2 * i, 0)
            stage(2 * i + 1, 1)
            return 0
        lax.fori_loop(0, (n_chunks - 1) // 2, pair, 0)

        wait_gather(0)
        wait_exps(0)
        compute(n_chunks - 1, 0)
        wait_exps(0)
        wait_exps(1)
        wait_rsum(0)
        wait_rsum(1)

        plsc.subcore_barrier()

        @pl.when(sid == 0)
        def _():
            pltpu.sync_copy(rs_sh, rowsum_h.at[cid])

    return body(embs, r0, r1)


def _pass_b(embs, r0, r1, exps, rowsum_p, n_pad, C):
    E = r0.shape[0]
    D = embs.shape[1]
    e_per_w = E // NW
    n_chunks = e_per_w // C
    assert n_chunks % 2 == 1 and n_chunks >= 3
    rows_per_tile = n_pad // NS
    mesh = plsc.VectorSubcoreMesh(core_axis_name="c", subcore_axis_name="s")

    @functools.partial(
        pl.kernel,
        out_type=jax.ShapeDtypeStruct((NC, n_pad, D), jnp.float32),
        mesh=mesh,
        compiler_params=pltpu.CompilerParams(needs_layout_passes=False),
        scratch_types=[
            pltpu.VMEM((C,), jnp.int32), pltpu.VMEM((C,), jnp.int32),
            pltpu.VMEM((C,), jnp.int32), pltpu.VMEM((C,), jnp.int32),
            pltpu.VMEM((C,), jnp.int32), pltpu.VMEM((C,), jnp.int32),
            pltpu.VMEM((C,), jnp.float32), pltpu.VMEM((C,), jnp.float32),
            pltpu.VMEM((C, D), jnp.float32), pltpu.VMEM((C, D), jnp.float32),
            pltpu.VMEM((n_pad,), jnp.float32),
            pltpu.VMEM((1024,), jnp.float32),
            pltpu.VMEM((1024,), jnp.float32),
            pltpu.VMEM((C, D), jnp.float32),
            pltpu.VMEM_SHARED((n_pad, D), jnp.float32),
        ] + [pltpu.SemaphoreType.DMA] * 10,
    )
    def body(embs_h, r0_h, r1_h, exps_h, rowsum_h, outp_h,
             idx0_0, idx0_1, idx1_0, idx1_1, sidx_0, sidx_1, w_0, w_1,
             b_0, b_1, rsum_v, s0_v, s1_v, z_v, out_sh,
             si0_0, si0_1, si1_0, si1_1, sw_0, sw_1, sg_0, sg_1,
             ssc_0, ssc_1):
        idx0 = [idx0_0, idx0_1]
        idx1 = [idx1_0, idx1_1]
        sidx = [sidx_0, sidx_1]
        w_v = [w_0, w_1]
        b_v = [b_0, b_1]
        si0 = [si0_0, si0_1]
        si1 = [si1_0, si1_1]
        sw = [sw_0, sw_1]
        sg = [sg_0, sg_1]
        ssc = [ssc_0, ssc_1]

        cid = lax.axis_index("c")
        sid = lax.axis_index("s")
        wid = sid * NC + cid
        lane = lax.broadcasted_iota(jnp.int32, (L,), 0)
        zero16 = jnp.zeros((L,), jnp.float32)

        def cbase(ci):
            return pl.multiple_of(wid * e_per_w + ci * C, 8)

        def issue_in(ci, p):
            base = cbase(ci)
            pltpu.async_copy(r0_h.at[pl.ds(base, C)], idx0[p], si0[p])
            pltpu.async_copy(r1_h.at[pl.ds(base, C)], idx1[p], si1[p])
            pltpu.async_copy(exps_h.at[pl.ds(base, C)], w_v[p], sw[p])

        def wait_in(p):
            pltpu.make_async_copy(r0_h.at[pl.ds(0, C)], idx0[p], si0[p]).wait()
            pltpu.make_async_copy(r1_h.at[pl.ds(0, C)], idx1[p], si1[p]).wait()
            pltpu.make_async_copy(exps_h.at[pl.ds(0, C)], w_v[p], sw[p]).wait()

        def issue_gather(p):
            pltpu.async_copy(embs_h.at[idx1[p]], b_v[p], sg[p])

        def wait_gather(p):
            pltpu.make_async_copy(embs_h.at[idx1[p]], b_v[p], sg[p]).wait()

        def wait_scat(p):
            pltpu.make_async_copy(b_v[p], out_sh.at[sidx[p]], ssc[p]).wait()

        def compute(p):
            def scale_g(g, _):
                i0 = idx0[p][pl.ds(g * L, L)]
                sidx[p][pl.ds(g * L, L)] = i0
                s16 = plsc.load_gather(rsum_v, [i0])
                w16 = w_v[p][pl.ds(g * L, L)] / s16
                for k in range(L):
                    e = g * L + k
                    w = w16[k]
                    for j in range(D // L):
                        b_v[p][e, pl.ds(j * L, L)] = (
                            b_v[p][e, pl.ds(j * L, L)] * w)
                return 0
            lax.fori_loop(0, C // L, scale_g, 0)
            pltpu.async_copy(b_v[p], out_sh.at[sidx[p]], ssc[p], add=True)

        issue_in(0, 0)
        issue_in(1, 1)

        def zrow(r, _):
            for j in range(D // L):
                z_v[r, pl.ds(j * L, L)] = zero16
            return 0
        lax.fori_loop(0, C, zrow, 0)
        for k in range(rows_per_tile // C):
            pltpu.sync_copy(z_v, out_sh.at[pl.ds(sid * rows_per_tile + k * C, C)])

        def rsblk(i, _):
            pltpu.sync_copy(rowsum_h.at[0, pl.ds(i * 1024, 1024)], s0_v)
            pltpu.sync_copy(rowsum_h.at[1, pl.ds(i * 1024, 1024)], s1_v)

            def add16(j, _):
                rsum_v[pl.ds(i * 1024 + j * L, L)] = (
                    s0_v[pl.ds(j * L, L)] + s1_v[pl.ds(j * L, L)])
                return 0
            lax.fori_loop(0, 1024 // L, add16, 0)
            return 0
        lax.fori_loop(0, n_pad // 1024, rsblk, 0)
        plsc.subcore_barrier()

        wait_in(0)
        issue_gather(0)

        def stage(ci, p):
            q = 1 - p

            @pl.when(ci + 1 < n_chunks)
            def _():
                wait_in(q)

                @pl.when(ci >= 1)
                def _():
                    wait_scat(q)
                issue_gather(q)
            wait_gather(p)
            compute(p)

            @pl.when(ci + 2 < n_chunks)
            def _():
                issue_in(ci + 2, p)

        def pair(i, _):
            stage(2 * i, 0)
            stage(2 * i + 1, 1)
            return 0
        lax.fori_loop(0, (n_chunks - 1) // 2, pair, 0)

        wait_gather(0)
        compute(0)
        wait_scat(1)
        wait_scat(0)

        plsc.subcore_barrier()
        pltpu.sync_copy(out_sh.at[pl.ds(sid * rows_per_tile, rows_per_tile)],
                        outp_h.at[cid, pl.ds(sid * rows_per_tile, rows_per_tile)])

    return body(embs, r0, r1, exps, rowsum_p)


def _combine(p0, p1):
    N, D = p0.shape
    BS = 400

    def body(a_ref, b_ref, o_ref):
        o_ref[...] = a_ref[...] + b_ref[...]

    return pl.pallas_call(
        body,
        grid=(N // BS,),
        in_specs=[pl.BlockSpec((BS, D), lambda i: (i, 0)),
                  pl.BlockSpec((BS, D), lambda i: (i, 0))],
        out_specs=pl.BlockSpec((BS, D), lambda i: (i, 0)),
        out_shape=jax.ShapeDtypeStruct((N, D), jnp.float32),
    )(p0, p1)


def kernel(embs, ratings, node_num):
    del node_num
    N, D = embs.shape
    E = ratings.shape[0]
    assert E % NW == 0 and D % L == 0
    C = 80
    n_pad = ((N + 1023) // 1024) * 1024
    r0 = ratings[:, 0].astype(jnp.int32)
    r1 = ratings[:, 1].astype(jnp.int32)
    exps, rowsum_p = _pass_a(embs, r0, r1, n_pad, C)
    outp = _pass_b(embs, r0, r1, exps, rowsum_p, n_pad, C)
    return _combine(outp[0, :N], outp[1, :N])
